# Initial kernel scaffold; baseline (speedup 1.0000x reference)
import functools

import jax
import jax.numpy as jnp
from jax import lax
from jax.experimental import pallas as pl
from jax.experimental.pallas import tpu as pltpu
from jax.experimental.pallas import tpu_sc as plsc

N = 10000
E = 160000
D_IN = 128
D_H = 256
NUM_MASK = 3000
NUM_NOISE = 300
NUM_TOKEN = 2700

NC = 2
NS = 16
L = 16

_F32 = jnp.float32


def _sc_degrees(src, dst, zeros16):
    B = 80
    EPT = E // NS
    NCH = EPT // B
    ROWS = N // NS
    mesh = plsc.VectorSubcoreMesh(core_axis_name="c", subcore_axis_name="s")

    @functools.partial(
        pl.kernel, mesh=mesh,
        out_type=(jax.ShapeDtypeStruct((N, L), _F32),
                  jax.ShapeDtypeStruct((N, L), _F32)),
        scratch_types=[
            pltpu.VMEM((B,), jnp.int32),
            pltpu.VMEM((B, L), _F32),
            pltpu.VMEM_SHARED((N, L), _F32),
        ],
    )
    def k(src_h, dst_h, z_h, o_src, o_dst, idxv, onesv, acc):
        c = lax.axis_index("c")
        s = lax.axis_index("s")
        rb = s * ROWS
        def fill_ones(r, _):
            onesv[r] = jnp.ones((L,), _F32)
            return ()
        lax.fori_loop(0, B, fill_ones, ())
        pltpu.sync_copy(z_h.at[pl.ds(rb, ROWS)], acc.at[pl.ds(rb, ROWS)])
        plsc.subcore_barrier()

        def body(i, _):
            off = s * EPT + i * B
            @pl.when(c == 0)
            def _():
                pltpu.sync_copy(src_h.at[pl.ds(off, B)], idxv)
            @pl.when(c == 1)
            def _():
                pltpu.sync_copy(dst_h.at[pl.ds(off, B)], idxv)
            pltpu.sync_copy(onesv, acc.at[idxv], add=True)
            return ()
        lax.fori_loop(0, NCH, body, ())
        plsc.subcore_barrier()

        @pl.when(c == 0)
        def _():
            pltpu.sync_copy(acc.at[pl.ds(rb, ROWS)], o_src.at[pl.ds(rb, ROWS)])
        @pl.when(c == 1)
        def _():
            pltpu.sync_copy(acc.at[pl.ds(rb, ROWS)], o_dst.at[pl.ds(rb, ROWS)])

    return k(src, dst, zeros16)


def _sc_mp2(t0, t1, src, dst, zeros):
    B = 80
    EPT = E // NS
    NCH = EPT // B
    ROWS = N // NS
    mesh = plsc.VectorSubcoreMesh(core_axis_name="c", subcore_axis_name="s")

    @functools.partial(
        pl.kernel, mesh=mesh,
        out_type=(jax.ShapeDtypeStruct((N, D_IN), _F32),
                  jax.ShapeDtypeStruct((N, D_IN), _F32)),
        scratch_types=[
            pltpu.VMEM((B,), jnp.int32),
            pltpu.VMEM((B,), jnp.int32),
            pltpu.VMEM((B, D_IN), _F32),
            pltpu.VMEM_SHARED((N, D_IN), _F32),
            pltpu.SemaphoreType.DMA,
        ],
    )
    def k(t0_h, t1_h, src_h, dst_h, z_h, o0, o1, sidx, didx, rows, acc, sem):
        c = lax.axis_index("c")
        s = lax.axis_index("s")
        rb = s * ROWS
        pltpu.sync_copy(z_h.at[pl.ds(rb, ROWS)], acc.at[pl.ds(rb, ROWS)])
        plsc.subcore_barrier()

        def loop(table_h):
            def body(i, _):
                off = s * EPT + i * B
                pltpu.sync_copy(src_h.at[pl.ds(off, B)], sidx)
                pltpu.sync_copy(dst_h.at[pl.ds(off, B)], didx)
                pltpu.async_copy(table_h.at[sidx], rows, sem).wait()
                pltpu.sync_copy(rows, acc.at[didx], add=True)
                return ()
            lax.fori_loop(0, NCH, body, ())

        @pl.when(c == 0)
        def _():
            loop(t0_h)
        @pl.when(c == 1)
        def _():
            loop(t1_h)
        plsc.subcore_barrier()

        @pl.when(c == 0)
        def _():
            pltpu.sync_copy(acc.at[pl.ds(rb, ROWS)], o0.at[pl.ds(rb, ROWS)])
        @pl.when(c == 1)
        def _():
            pltpu.sync_copy(acc.at[pl.ds(rb, ROWS)], o1.at[pl.ds(rb, ROWS)])

    return k(t0, t1, src, dst, zeros)


def _sc_mp1(t, src, dst, zeros):
    B = 40
    EPC = E // NC
    EPT = EPC // NS
    NCH = EPT // B
    ROWS = N // NS
    mesh = plsc.VectorSubcoreMesh(core_axis_name="c", subcore_axis_name="s")

    @functools.partial(
        pl.kernel, mesh=mesh,
        out_type=(jax.ShapeDtypeStruct((N, D_IN), _F32),
                  jax.ShapeDtypeStruct((N, D_IN), _F32)),
        scratch_types=[
            pltpu.VMEM((B,), jnp.int32),
            pltpu.VMEM((B,), jnp.int32),
            pltpu.VMEM((B, D_IN), _F32),
            pltpu.VMEM_SHARED((N, D_IN), _F32),
            pltpu.SemaphoreType.DMA,
        ],
    )
    def k(t_h, src_h, dst_h, z_h, o0, o1, sidx, didx, rows, acc, sem):
        c = lax.axis_index("c")
        s = lax.axis_index("s")
        rb = s * ROWS
        pltpu.sync_copy(z_h.at[pl.ds(rb, ROWS)], acc.at[pl.ds(rb, ROWS)])
        plsc.subcore_barrier()

        def body(i, _):
            off = c * EPC + s * EPT + i * B
            pltpu.sync_copy(src_h.at[pl.ds(off, B)], sidx)
            pltpu.sync_copy(dst_h.at[pl.ds(off, B)], didx)
            pltpu.async_copy(t_h.at[sidx], rows, sem).wait()
            pltpu.sync_copy(rows, acc.at[didx], add=True)
            return ()
        lax.fori_loop(0, NCH, body, ())
        plsc.subcore_barrier()

        @pl.when(c == 0)
        def _():
            pltpu.sync_copy(acc.at[pl.ds(rb, ROWS)], o0.at[pl.ds(rb, ROWS)])
        @pl.when(c == 1)
        def _():
            pltpu.sync_copy(acc.at[pl.ds(rb, ROWS)], o1.at[pl.ds(rb, ROWS)])

    return k(t, src, dst, zeros)


def _sc_edge_abs(xarr, recon, src, dst):
    B = 40
    EPC = E // NC
    EPT = EPC // NS
    NCH = EPT // B
    mesh = plsc.VectorSubcoreMesh(core_axis_name="c", subcore_axis_name="s")

    @functools.partial(
        pl.kernel, mesh=mesh,
        out_type=(jax.ShapeDtypeStruct((E, D_IN), _F32),
                  jax.ShapeDtypeStruct((E, D_IN), _F32)),
        scratch_types=[
            pltpu.VMEM((B,), jnp.int32),
            pltpu.VMEM((B,), jnp.int32),
            pltpu.VMEM((B, D_IN), _F32),
            pltpu.VMEM((B, D_IN), _F32),
            pltpu.VMEM((B, D_IN), _F32),
            pltpu.SemaphoreType.DMA,
        ],
    )
    def k(x_h, r_h, src_h, dst_h, dxi_h, dxr_h, sidx, didx, ba, bb, bo, sem):
        c = lax.axis_index("c")
        s = lax.axis_index("s")

        def absdiff():
            def rowbody(r, _):
                for j in range(D_IN // L):
                    sl = pl.ds(j * L, L)
                    bo[r, sl] = jnp.abs(ba[r, sl] - bb[r, sl])
                return ()
            lax.fori_loop(0, B, rowbody, ())

        def body(i, _):
            off = c * EPC + s * EPT + i * B
            pltpu.sync_copy(src_h.at[pl.ds(off, B)], sidx)
            pltpu.sync_copy(dst_h.at[pl.ds(off, B)], didx)
            pltpu.async_copy(x_h.at[sidx], ba, sem).wait()
            pltpu.async_copy(x_h.at[didx], bb, sem).wait()
            absdiff()
            pltpu.sync_copy(bo, dxi_h.at[pl.ds(off, B)])
            pltpu.async_copy(r_h.at[sidx], ba, sem).wait()
            pltpu.async_copy(r_h.at[didx], bb, sem).wait()
            absdiff()
            pltpu.sync_copy(bo, dxr_h.at[pl.ds(off, B)])
            return ()
        lax.fori_loop(0, NCH, body, ())

    return k(xarr, recon, src, dst)


_BN = 1000
_BE = 1000


def _rsqrt_deg(d):
    return lax.rsqrt(jnp.maximum(d, 1.0))


def _tc1_body(x_ref, w_ref, dout_ref, kt_ref, o0_ref, o1_ref):
    cs = _rsqrt_deg(dout_ref[...])
    t = jnp.dot(x_ref[...], w_ref[...], preferred_element_type=_F32)
    t = t * (kt_ref[...] * cs)
    o0_ref[...] = t[:, :D_IN]
    o1_ref[...] = t[:, D_IN:]


def _tc1(x_used, w1, deg_out, kt):
    return pl.pallas_call(
        _tc1_body,
        grid=(N // _BN,),
        in_specs=[
            pl.BlockSpec((_BN, D_IN), lambda i: (i, 0)),
            pl.BlockSpec((D_IN, D_H), lambda i: (0, 0)),
            pl.BlockSpec((_BN, 1), lambda i: (i, 0)),
            pl.BlockSpec((_BN, 1), lambda i: (i, 0)),
        ],
        out_specs=[pl.BlockSpec((_BN, D_IN), lambda i: (i, 0))] * 2,
        out_shape=[jax.ShapeDtypeStruct((N, D_IN), _F32)] * 2,
    )(x_used, w1, deg_out, kt)


def _tc2_body(a0_ref, a1_ref, din_ref, dout_ref, b_ref, al_ref, w_ref,
              o0_ref, o1_ref):
    cd = _rsqrt_deg(din_ref[...])
    cs = _rsqrt_deg(dout_ref[...])
    h = jnp.concatenate([a0_ref[...], a1_ref[...]], axis=1) * cd + b_ref[...]
    a = al_ref[0, 0]
    h = jnp.where(h > 0, h, a * h)
    t = jnp.dot(h, w_ref[...], preferred_element_type=_F32) * cs
    o0_ref[...] = t[:, :D_IN]
    o1_ref[...] = t[:, D_IN:]


def _tc2(a0, a1, deg_in, deg_out, b, alpha, w):
    return pl.pallas_call(
        _tc2_body,
        grid=(N // _BN,),
        in_specs=[
            pl.BlockSpec((_BN, D_IN), lambda i: (i, 0)),
            pl.BlockSpec((_BN, D_IN), lambda i: (i, 0)),
            pl.BlockSpec((_BN, 1), lambda i: (i, 0)),
            pl.BlockSpec((_BN, 1), lambda i: (i, 0)),
            pl.BlockSpec((1, D_H), lambda i: (0, 0)),
            pl.BlockSpec((1, 1), lambda i: (0, 0)),
            pl.BlockSpec((D_H, D_H), lambda i: (0, 0)),
        ],
        out_specs=[pl.BlockSpec((_BN, D_IN), lambda i: (i, 0))] * 2,
        out_shape=[jax.ShapeDtypeStruct((N, D_IN), _F32)] * 2,
    )(a0, a1, deg_in, deg_out, b, alpha, w)


def _tc3_body(a0_ref, a1_ref, din_ref, dout_ref, b_ref, al_ref, we_ref,
              wd_ref, mk_ref, o_ref):
    cd = _rsqrt_deg(din_ref[...])
    cs = _rsqrt_deg(dout_ref[...])
    h = jnp.concatenate([a0_ref[...], a1_ref[...]], axis=1) * cd + b_ref[...]
    a = al_ref[0, 0]
    h = jnp.where(h > 0, h, a * h)
    rep = jnp.dot(h, we_ref[...], preferred_element_type=_F32) * mk_ref[...]
    o_ref[...] = jnp.dot(rep, wd_ref[...], preferred_element_type=_F32) * cs


def _tc3(a0, a1, deg_in, deg_out, b, alpha, w_e2d, w_dec, maskkeep):
    return pl.pallas_call(
        _tc3_body,
        grid=(N // _BN,),
        in_specs=[
            pl.BlockSpec((_BN, D_IN), lambda i: (i, 0)),
            pl.BlockSpec((_BN, D_IN), lambda i: (i, 0)),
            pl.BlockSpec((_BN, 1), lambda i: (i, 0)),
            pl.BlockSpec((_BN, 1), lambda i: (i, 0)),
            pl.BlockSpec((1, D_H), lambda i: (0, 0)),
            pl.BlockSpec((1, 1), lambda i: (0, 0)),
            pl.BlockSpec((D_H, D_H), lambda i: (0, 0)),
            pl.BlockSpec((D_H, D_IN), lambda i: (0, 0)),
            pl.BlockSpec((_BN, 1), lambda i: (i, 0)),
        ],
        out_specs=pl.BlockSpec((_BN, D_IN), lambda i: (i, 0)),
        out_shape=jax.ShapeDtypeStruct((N, D_IN), _F32),
    )(a0, a1, deg_in, deg_out, b, alpha, w_e2d, w_dec, maskkeep)


def _tc4_body(p0_ref, p1_ref, din_ref, b_ref, x_ref, ms_ref,
              recon_ref, acc_ref):
    i = pl.program_id(0)
    cd = _rsqrt_deg(din_ref[...])
    r = (p0_ref[...] + p1_ref[...]) * cd + b_ref[...]
    recon_ref[...] = r
    xb = x_ref[...]
    nr = jnp.sqrt(jnp.sum(r * r, axis=1, keepdims=True)) + 1e-8
    nx = jnp.sqrt(jnp.sum(xb * xb, axis=1, keepdims=True)) + 1e-8
    dots = jnp.sum(r * xb, axis=1, keepdims=True)
    term = (1.0 - dots / (nr * nx)) ** 2
    val = jnp.sum(term * ms_ref[...])
    @pl.when(i == 0)
    def _():
        acc_ref[0, 0] = 0.0
    acc_ref[0, 0] += val


def _tc4(p0, p1, deg_in, b_dec, x, masksel):
    return pl.pallas_call(
        _tc4_body,
        grid=(N // _BN,),
        in_specs=[
            pl.BlockSpec((_BN, D_IN), lambda i: (i, 0)),
            pl.BlockSpec((_BN, D_IN), lambda i: (i, 0)),
            pl.BlockSpec((_BN, 1), lambda i: (i, 0)),
            pl.BlockSpec((1, D_IN), lambda i: (0, 0)),
            pl.BlockSpec((_BN, D_IN), lambda i: (i, 0)),
            pl.BlockSpec((_BN, 1), lambda i: (i, 0)),
        ],
        out_specs=[
            pl.BlockSpec((_BN, D_IN), lambda i: (i, 0)),
            pl.BlockSpec((1, 1), lambda i: (0, 0)),
        ],
        out_shape=[
            jax.ShapeDtypeStruct((N, D_IN), _F32),
            jax.ShapeDtypeStruct((1, 1), _F32),
        ],
    )(p0, p1, deg_in, b_dec, x, masksel)


def _tc5_body(u_ref, v_ref, acc_ref):
    i = pl.program_id(0)
    u = u_ref[...]
    v = v_ref[...]
    mu = jnp.max(u, axis=1, keepdims=True)
    mv = jnp.max(v, axis=1, keepdims=True)
    eu = jnp.exp(u - mu)
    su = jnp.sum(eu, axis=1, keepdims=True)
    sv = jnp.sum(jnp.exp(v - mv), axis=1, keepdims=True)
    t = jnp.sum(eu * (u - v), axis=1, keepdims=True)
    row = t / su - mu - jnp.log(su) + mv + jnp.log(sv)
    val = jnp.sum(row)
    @pl.when(i == 0)
    def _():
        acc_ref[0, 0] = 0.0
    acc_ref[0, 0] += val


def _tc5(dxi, dxr):
    return pl.pallas_call(
        _tc5_body,
        grid=(E // _BE,),
        in_specs=[
            pl.BlockSpec((_BE, D_IN), lambda i: (i, 0)),
            pl.BlockSpec((_BE, D_IN), lambda i: (i, 0)),
        ],
        out_specs=pl.BlockSpec((1, 1), lambda i: (0, 0)),
        out_shape=jax.ShapeDtypeStruct((1, 1), _F32),
    )(dxi, dxr)


def kernel(x, edge_index, A, W1, b1, a1, W2, b2, a2, W_e2d, W_dec, b_dec):
    del A
    src = edge_index[0]
    dst = edge_index[1]

    mk = jax.random.key(42)
    k1, k2, k3 = jax.random.split(mk, 3)
    perm = jax.random.permutation(k1, N)
    mask_nodes = perm[:NUM_MASK]
    perm_mask = jax.random.permutation(k2, NUM_MASK)
    token_nodes = mask_nodes[perm_mask[:NUM_TOKEN]]
    noise_nodes = mask_nodes[perm_mask[-NUM_NOISE:]]
    noise_chosen = jax.random.permutation(k3, N)[:NUM_NOISE]

    ones_n = jnp.ones((N, 1), _F32)
    kt = ones_n.at[token_nodes].set(0.0)
    maskkeep = ones_n.at[mask_nodes].set(0.0)
    masksel = 1.0 - maskkeep
    x_used = x.at[noise_nodes].set(x[noise_chosen])

    zeros_rows = jnp.zeros((N, D_IN), _F32)
    zeros16 = jnp.zeros((N, L), _F32)

    dsrc16, ddst16 = _sc_degrees(src, dst, zeros16)
    deg_out = dsrc16[:, :1]
    deg_in = ddst16[:, :1]

    a1_s = a1.reshape(1, 1)
    a2_s = a2.reshape(1, 1)
    b1_r = b1.reshape(1, D_H)
    b2_r = b2.reshape(1, D_H)
    bd_r = b_dec.reshape(1, D_IN)

    t0, t1 = _tc1(x_used, W1, deg_out, kt)
    g0, g1 = _sc_mp2(t0, t1, src, dst, zeros_rows)
    u0, u1 = _tc2(g0, g1, deg_in, deg_out, b1_r, a1_s, W2)
    h0, h1 = _sc_mp2(u0, u1, src, dst, zeros_rows)
    t3 = _tc3(h0, h1, deg_in, deg_out, b2_r, a2_s, W_e2d, W_dec, maskkeep)
    p0, p1 = _sc_mp1(t3, src, dst, zeros_rows)
    recon, loss_acc = _tc4(p0, p1, deg_in, bd_r, x, masksel)
    loss = loss_acc[0, 0] / NUM_MASK

    dxi, dxr = _sc_edge_abs(x, recon, src, dst)
    kl_acc = _tc5(dxi, dxr)
    loss_s = kl_acc[0, 0] / E

    return (loss, loss_s, recon)


# speedup vs baseline: 2.9253x; 2.9253x over previous
import functools

import jax
import jax.numpy as jnp
from jax import lax
from jax.experimental import pallas as pl
from jax.experimental.pallas import tpu as pltpu
from jax.experimental.pallas import tpu_sc as plsc

N = 10000
E = 160000
D_IN = 128
D_H = 256
NUM_MASK = 3000
NUM_NOISE = 300
NUM_TOKEN = 2700

NC = 2
NS = 16
L = 16

_F32 = jnp.float32

_RT = 624


def _tile_rows(s, fn):
    @pl.when(s < NS - 1)
    def _():
        fn(s * _RT, _RT)
    @pl.when(s == NS - 1)
    def _():
        fn((NS - 1) * _RT, N - (NS - 1) * _RT)


def _sc_degrees(ei_flat, zeros):
    B = 80
    EPT = E // NS
    NCH = EPT // B
    mesh = plsc.VectorSubcoreMesh(core_axis_name="c", subcore_axis_name="s")

    @functools.partial(
        pl.kernel, mesh=mesh,
        out_type=jax.ShapeDtypeStruct((2 * N, D_IN), _F32),
        scratch_types=[
            pltpu.VMEM((B,), jnp.int32),
            pltpu.VMEM((B, D_IN), _F32),
            pltpu.VMEM_SHARED((N, D_IN), _F32),
        ],
    )
    def k(ei_h, z_h, out, idxv, onesv, acc):
        c = lax.axis_index("c")
        s = lax.axis_index("s")
        def fill_ones(r, _):
            for j in range(D_IN // L):
                onesv[r, pl.ds(j * L, L)] = jnp.ones((L,), _F32)
            return ()
        lax.fori_loop(0, B, fill_ones, ())
        _tile_rows(s, lambda rb, nr: pltpu.sync_copy(
            z_h.at[pl.ds(rb, nr)], acc.at[pl.ds(rb, nr)]))
        plsc.subcore_barrier()

        def body(i, _):
            off = c * E + s * EPT + i * B
            pltpu.sync_copy(ei_h.at[pl.ds(off, B)], idxv)
            pltpu.sync_copy(onesv, acc.at[idxv], add=True)
            return ()
        lax.fori_loop(0, NCH, body, ())
        plsc.subcore_barrier()

        _tile_rows(s, lambda rb, nr: pltpu.sync_copy(
            acc.at[pl.ds(rb, nr)], out.at[pl.ds(c * N + rb, nr)]))

    return k(ei_flat, zeros)


def _sc_mp2(t2n, src, dst, zeros):
    B = 80
    EPT = E // NS
    NCH = EPT // B
    mesh = plsc.VectorSubcoreMesh(core_axis_name="c", subcore_axis_name="s")

    @functools.partial(
        pl.kernel, mesh=mesh,
        out_type=jax.ShapeDtypeStruct((2 * N, D_IN), _F32),
        scratch_types=[
            pltpu.VMEM((B,), jnp.int32),
            pltpu.VMEM((B,), jnp.int32),
            pltpu.VMEM((B, D_IN), _F32),
            pltpu.VMEM_SHARED((N, D_IN), _F32),
            pltpu.SemaphoreType.DMA,
        ],
    )
    def k(t_h, src_h, dst_h, z_h, out, sidx, didx, rows, acc, sem):
        c = lax.axis_index("c")
        s = lax.axis_index("s")
        _tile_rows(s, lambda rb, nr: pltpu.sync_copy(
            z_h.at[pl.ds(rb, nr)], acc.at[pl.ds(rb, nr)]))
        plsc.subcore_barrier()

        rowoff = c * N

        def body(i, _):
            off = s * EPT + i * B
            pltpu.sync_copy(src_h.at[pl.ds(off, B)], sidx)
            pltpu.sync_copy(dst_h.at[pl.ds(off, B)], didx)
            for j in range(B // L):
                sl = pl.ds(j * L, L)
                sidx[sl] = sidx[sl] + rowoff
            pltpu.async_copy(t_h.at[sidx], rows, sem).wait()
            pltpu.sync_copy(rows, acc.at[didx], add=True)
            return ()
        lax.fori_loop(0, NCH, body, ())
        plsc.subcore_barrier()

        _tile_rows(s, lambda rb, nr: pltpu.sync_copy(
            acc.at[pl.ds(rb, nr)], out.at[pl.ds(c * N + rb, nr)]))

    return k(t2n, src, dst, zeros)


def _sc_mp1(t, src, dst, zeros):
    B = 40
    EPC = E // NC
    EPT = EPC // NS
    NCH = EPT // B
    mesh = plsc.VectorSubcoreMesh(core_axis_name="c", subcore_axis_name="s")

    @functools.partial(
        pl.kernel, mesh=mesh,
        out_type=jax.ShapeDtypeStruct((2 * N, D_IN), _F32),
        scratch_types=[
            pltpu.VMEM((B,), jnp.int32),
            pltpu.VMEM((B,), jnp.int32),
            pltpu.VMEM((B, D_IN), _F32),
            pltpu.VMEM_SHARED((N, D_IN), _F32),
            pltpu.SemaphoreType.DMA,
        ],
    )
    def k(t_h, src_h, dst_h, z_h, out, sidx, didx, rows, acc, sem):
        c = lax.axis_index("c")
        s = lax.axis_index("s")
        _tile_rows(s, lambda rb, nr: pltpu.sync_copy(
            z_h.at[pl.ds(rb, nr)], acc.at[pl.ds(rb, nr)]))
        plsc.subcore_barrier()

        def body(i, _):
            off = c * EPC + s * EPT + i * B
            pltpu.sync_copy(src_h.at[pl.ds(off, B)], sidx)
            pltpu.sync_copy(dst_h.at[pl.ds(off, B)], didx)
            pltpu.async_copy(t_h.at[sidx], rows, sem).wait()
            pltpu.sync_copy(rows, acc.at[didx], add=True)
            return ()
        lax.fori_loop(0, NCH, body, ())
        plsc.subcore_barrier()

        _tile_rows(s, lambda rb, nr: pltpu.sync_copy(
            acc.at[pl.ds(rb, nr)], out.at[pl.ds(c * N + rb, nr)]))

    return k(t, src, dst, zeros)


def _sc_edge_abs(xarr, recon, src, dst):
    B = 40
    EPC = E // NC
    EPT = EPC // NS
    NCH = EPT // B
    mesh = plsc.VectorSubcoreMesh(core_axis_name="c", subcore_axis_name="s")

    @functools.partial(
        pl.kernel, mesh=mesh,
        out_type=(jax.ShapeDtypeStruct((E, D_IN), _F32),
                  jax.ShapeDtypeStruct((E, D_IN), _F32)),
        scratch_types=[
            pltpu.VMEM((B,), jnp.int32),
            pltpu.VMEM((B,), jnp.int32),
            pltpu.VMEM((B, D_IN), _F32),
            pltpu.VMEM((B, D_IN), _F32),
            pltpu.VMEM((B, D_IN), _F32),
            pltpu.SemaphoreType.DMA,
        ],
    )
    def k(x_h, r_h, src_h, dst_h, dxi_h, dxr_h, sidx, didx, ba, bb, bo, sem):
        c = lax.axis_index("c")
        s = lax.axis_index("s")

        def absdiff():
            def rowbody(r, _):
                for j in range(D_IN // L):
                    sl = pl.ds(j * L, L)
                    bo[r, sl] = jnp.abs(ba[r, sl] - bb[r, sl])
                return ()
            lax.fori_loop(0, B, rowbody, ())

        def body(i, _):
            off = c * EPC + s * EPT + i * B
            pltpu.sync_copy(src_h.at[pl.ds(off, B)], sidx)
            pltpu.sync_copy(dst_h.at[pl.ds(off, B)], didx)
            pltpu.async_copy(x_h.at[sidx], ba, sem).wait()
            pltpu.async_copy(x_h.at[didx], bb, sem).wait()
            absdiff()
            pltpu.sync_copy(bo, dxi_h.at[pl.ds(off, B)])
            pltpu.async_copy(r_h.at[sidx], ba, sem).wait()
            pltpu.async_copy(r_h.at[didx], bb, sem).wait()
            absdiff()
            pltpu.sync_copy(bo, dxr_h.at[pl.ds(off, B)])
            return ()
        lax.fori_loop(0, NCH, body, ())

    return k(xarr, recon, src, dst)


_BN = 1000
_BE = 1000


def _rsqrt_deg(d):
    return lax.rsqrt(jnp.maximum(d, 1.0))


def _tc1_body(x_ref, w_ref, dout_ref, kt_ref, o_ref):
    cs = _rsqrt_deg(dout_ref[...])
    t = jnp.dot(x_ref[...], w_ref[...], preferred_element_type=_F32)
    t = t * (kt_ref[...] * cs)
    o_ref[0] = t[:, :D_IN]
    o_ref[1] = t[:, D_IN:]


def _tc1(x_used, w1, deg_out, kt):
    return pl.pallas_call(
        _tc1_body,
        grid=(N // _BN,),
        in_specs=[
            pl.BlockSpec((_BN, D_IN), lambda i: (i, 0)),
            pl.BlockSpec((D_IN, D_H), lambda i: (0, 0)),
            pl.BlockSpec((_BN, 1), lambda i: (i, 0)),
            pl.BlockSpec((_BN, 1), lambda i: (i, 0)),
        ],
        out_specs=pl.BlockSpec((2, _BN, D_IN), lambda i: (0, i, 0)),
        out_shape=jax.ShapeDtypeStruct((2, N, D_IN), _F32),
    )(x_used, w1, deg_out, kt)


def _tc2_body(a0_ref, a1_ref, din_ref, dout_ref, b_ref, al_ref, w_ref,
              o_ref):
    cd = _rsqrt_deg(din_ref[...])
    cs = _rsqrt_deg(dout_ref[...])
    h = jnp.concatenate([a0_ref[...], a1_ref[...]], axis=1) * cd + b_ref[...]
    a = al_ref[0, 0]
    h = jnp.where(h > 0, h, a * h)
    t = jnp.dot(h, w_ref[...], preferred_element_type=_F32) * cs
    o_ref[0] = t[:, :D_IN]
    o_ref[1] = t[:, D_IN:]


def _tc2(a2n, deg_in, deg_out, b, alpha, w):
    nb = N // _BN
    return pl.pallas_call(
        _tc2_body,
        grid=(nb,),
        in_specs=[
            pl.BlockSpec((_BN, D_IN), lambda i: (i, 0)),
            pl.BlockSpec((_BN, D_IN), lambda i: (i + N // _BN, 0)),
            pl.BlockSpec((_BN, 1), lambda i: (i, 0)),
            pl.BlockSpec((_BN, 1), lambda i: (i, 0)),
            pl.BlockSpec((1, D_H), lambda i: (0, 0)),
            pl.BlockSpec((1, 1), lambda i: (0, 0)),
            pl.BlockSpec((D_H, D_H), lambda i: (0, 0)),
        ],
        out_specs=pl.BlockSpec((2, _BN, D_IN), lambda i: (0, i, 0)),
        out_shape=jax.ShapeDtypeStruct((2, N, D_IN), _F32),
    )(a2n, a2n, deg_in, deg_out, b, alpha, w)


def _tc3_body(a0_ref, a1_ref, din_ref, dout_ref, b_ref, al_ref, we_ref,
              wd_ref, mk_ref, o_ref):
    cd = _rsqrt_deg(din_ref[...])
    cs = _rsqrt_deg(dout_ref[...])
    h = jnp.concatenate([a0_ref[...], a1_ref[...]], axis=1) * cd + b_ref[...]
    a = al_ref[0, 0]
    h = jnp.where(h > 0, h, a * h)
    rep = jnp.dot(h, we_ref[...], preferred_element_type=_F32) * mk_ref[...]
    o_ref[...] = jnp.dot(rep, wd_ref[...], preferred_element_type=_F32) * cs


def _tc3(a2n, deg_in, deg_out, b, alpha, w_e2d, w_dec, maskkeep):
    return pl.pallas_call(
        _tc3_body,
        grid=(N // _BN,),
        in_specs=[
            pl.BlockSpec((_BN, D_IN), lambda i: (i, 0)),
            pl.BlockSpec((_BN, D_IN), lambda i: (i + N // _BN, 0)),
            pl.BlockSpec((_BN, 1), lambda i: (i, 0)),
            pl.BlockSpec((_BN, 1), lambda i: (i, 0)),
            pl.BlockSpec((1, D_H), lambda i: (0, 0)),
            pl.BlockSpec((1, 1), lambda i: (0, 0)),
            pl.BlockSpec((D_H, D_H), lambda i: (0, 0)),
            pl.BlockSpec((D_H, D_IN), lambda i: (0, 0)),
            pl.BlockSpec((_BN, 1), lambda i: (i, 0)),
        ],
        out_specs=pl.BlockSpec((_BN, D_IN), lambda i: (i, 0)),
        out_shape=jax.ShapeDtypeStruct((N, D_IN), _F32),
    )(a2n, a2n, deg_in, deg_out, b, alpha, w_e2d, w_dec, maskkeep)


def _tc4_body(p0_ref, p1_ref, din_ref, b_ref, x_ref, ms_ref,
              recon_ref, acc_ref):
    i = pl.program_id(0)
    cd = _rsqrt_deg(din_ref[...])
    r = (p0_ref[...] + p1_ref[...]) * cd + b_ref[...]
    recon_ref[...] = r
    xb = x_ref[...]
    nr = jnp.sqrt(jnp.sum(r * r, axis=1, keepdims=True)) + 1e-8
    nx = jnp.sqrt(jnp.sum(xb * xb, axis=1, keepdims=True)) + 1e-8
    dots = jnp.sum(r * xb, axis=1, keepdims=True)
    term = (1.0 - dots / (nr * nx)) ** 2
    val = jnp.sum(term * ms_ref[...])
    @pl.when(i == 0)
    def _():
        acc_ref[...] = jnp.zeros((1, 1), _F32)
    acc_ref[...] = acc_ref[...] + val


def _tc4(p2n, deg_in, b_dec, x, masksel):
    return pl.pallas_call(
        _tc4_body,
        grid=(N // _BN,),
        in_specs=[
            pl.BlockSpec((_BN, D_IN), lambda i: (i, 0)),
            pl.BlockSpec((_BN, D_IN), lambda i: (i + N // _BN, 0)),
            pl.BlockSpec((_BN, 1), lambda i: (i, 0)),
            pl.BlockSpec((1, D_IN), lambda i: (0, 0)),
            pl.BlockSpec((_BN, D_IN), lambda i: (i, 0)),
            pl.BlockSpec((_BN, 1), lambda i: (i, 0)),
        ],
        out_specs=[
            pl.BlockSpec((_BN, D_IN), lambda i: (i, 0)),
            pl.BlockSpec((1, 1), lambda i: (0, 0)),
        ],
        out_shape=[
            jax.ShapeDtypeStruct((N, D_IN), _F32),
            jax.ShapeDtypeStruct((1, 1), _F32),
        ],
    )(p2n, p2n, deg_in, b_dec, x, masksel)


def _tc5_body(u_ref, v_ref, acc_ref):
    i = pl.program_id(0)
    u = u_ref[...]
    v = v_ref[...]
    mu = jnp.max(u, axis=1, keepdims=True)
    mv = jnp.max(v, axis=1, keepdims=True)
    eu = jnp.exp(u - mu)
    su = jnp.sum(eu, axis=1, keepdims=True)
    sv = jnp.sum(jnp.exp(v - mv), axis=1, keepdims=True)
    t = jnp.sum(eu * (u - v), axis=1, keepdims=True)
    row = t / su - mu - jnp.log(su) + mv + jnp.log(sv)
    val = jnp.sum(row)
    @pl.when(i == 0)
    def _():
        acc_ref[...] = jnp.zeros((1, 1), _F32)
    acc_ref[...] = acc_ref[...] + val


def _tc5(dxi, dxr):
    return pl.pallas_call(
        _tc5_body,
        grid=(E // _BE,),
        in_specs=[
            pl.BlockSpec((_BE, D_IN), lambda i: (i, 0)),
            pl.BlockSpec((_BE, D_IN), lambda i: (i, 0)),
        ],
        out_specs=pl.BlockSpec((1, 1), lambda i: (0, 0)),
        out_shape=jax.ShapeDtypeStruct((1, 1), _F32),
    )(dxi, dxr)


def kernel(x, edge_index, A, W1, b1, a1, W2, b2, a2, W_e2d, W_dec, b_dec):
    del A
    src = edge_index[0]
    dst = edge_index[1]

    mk = jax.random.key(42)
    k1, k2, k3 = jax.random.split(mk, 3)
    perm = jax.random.permutation(k1, N)
    mask_nodes = perm[:NUM_MASK]
    perm_mask = jax.random.permutation(k2, NUM_MASK)
    token_nodes = mask_nodes[perm_mask[:NUM_TOKEN]]
    noise_nodes = mask_nodes[perm_mask[-NUM_NOISE:]]
    noise_chosen = jax.random.permutation(k3, N)[:NUM_NOISE]

    ones_n = jnp.ones((N, 1), _F32)
    kt = ones_n.at[token_nodes].set(0.0)
    maskkeep = ones_n.at[mask_nodes].set(0.0)
    masksel = 1.0 - maskkeep
    x_used = x.at[noise_nodes].set(x[noise_chosen])

    zeros_rows = jnp.zeros((N, D_IN), _F32)

    deg2 = _sc_degrees(edge_index.reshape(2 * E), zeros_rows)
    deg_out = deg2[:N, :1]
    deg_in = deg2[N:, :1]

    a1_s = a1.reshape(1, 1)
    a2_s = a2.reshape(1, 1)
    b1_r = b1.reshape(1, D_H)
    b2_r = b2.reshape(1, D_H)
    bd_r = b_dec.reshape(1, D_IN)

    t2n = _tc1(x_used, W1, deg_out, kt).reshape(2 * N, D_IN)
    g2n = _sc_mp2(t2n, src, dst, zeros_rows)
    u2n = _tc2(g2n, deg_in, deg_out, b1_r, a1_s, W2).reshape(2 * N, D_IN)
    h2n = _sc_mp2(u2n, src, dst, zeros_rows)
    t3 = _tc3(h2n, deg_in, deg_out, b2_r, a2_s, W_e2d, W_dec, maskkeep)
    p2n = _sc_mp1(t3, src, dst, zeros_rows)
    recon, loss_acc = _tc4(p2n, deg_in, bd_r, x, masksel)
    loss = loss_acc[0, 0] / NUM_MASK

    dxi, dxr = _sc_edge_abs(x, recon, src, dst)
    kl_acc = _tc5(dxi, dxr)
    loss_s = kl_acc[0, 0] / E

    return (loss, loss_s, recon)


# speedup vs baseline: 4.9211x; 1.6822x over previous
import functools

import jax
import jax.numpy as jnp
from jax import lax
from jax.experimental import pallas as pl
from jax.experimental.pallas import tpu as pltpu
from jax.experimental.pallas import tpu_sc as plsc

N = 10000
E = 160000
D_IN = 128
D_H = 256
NUM_MASK = 3000
NUM_NOISE = 300
NUM_TOKEN = 2700

NC = 2
NS = 16
L = 16

_F32 = jnp.float32

_RT = 624


def _tile_rows(s, fn):
    @pl.when(s < NS - 1)
    def _():
        fn(s * _RT, _RT)
    @pl.when(s == NS - 1)
    def _():
        fn((NS - 1) * _RT, N - (NS - 1) * _RT)


def _sc_degrees(ei_flat, zeros):
    B = 80
    EPT = E // NS
    NCH = EPT // B
    mesh = plsc.VectorSubcoreMesh(core_axis_name="c", subcore_axis_name="s")

    @functools.partial(
        pl.kernel, mesh=mesh,
        out_type=jax.ShapeDtypeStruct((2 * N, D_IN), _F32),
        scratch_types=[
            pltpu.VMEM((EPT,), jnp.int32),
            pltpu.VMEM((B,), jnp.int32),
            pltpu.VMEM((B, D_IN), _F32),
            pltpu.VMEM_SHARED((N, D_IN), _F32),
        ],
    )
    def k(ei_h, z_h, out, eibuf, idxv, onesv, acc):
        c = lax.axis_index("c")
        s = lax.axis_index("s")
        pltpu.sync_copy(ei_h.at[pl.ds(c * E + s * EPT, EPT)], eibuf)
        def fill_ones(r, _):
            for j in range(D_IN // L):
                onesv[r, pl.ds(j * L, L)] = jnp.ones((L,), _F32)
            return ()
        lax.fori_loop(0, B, fill_ones, ())
        _tile_rows(s, lambda rb, nr: pltpu.sync_copy(
            z_h.at[pl.ds(rb, nr)], acc.at[pl.ds(rb, nr)]))
        plsc.subcore_barrier()

        def body(i, _):
            for j in range(B // L):
                idxv[pl.ds(j * L, L)] = eibuf[pl.ds(i * B + j * L, L)]
            pltpu.sync_copy(onesv, acc.at[idxv], add=True)
            return ()
        lax.fori_loop(0, NCH, body, ())
        plsc.subcore_barrier()

        _tile_rows(s, lambda rb, nr: pltpu.sync_copy(
            acc.at[pl.ds(rb, nr)], out.at[pl.ds(c * N + rb, nr)]))

    return k(ei_flat, zeros)


def _sc_mp2(t2n, src, dst, zeros):
    B = 80
    EPT = E // NS
    NCH = EPT // B
    mesh = plsc.VectorSubcoreMesh(core_axis_name="c", subcore_axis_name="s")

    @functools.partial(
        pl.kernel, mesh=mesh,
        out_type=jax.ShapeDtypeStruct((2 * N, D_IN), _F32),
        scratch_types=[
            pltpu.VMEM((EPT,), jnp.int32),
            pltpu.VMEM((EPT,), jnp.int32),
            pltpu.VMEM((B,), jnp.int32),
            pltpu.VMEM((B, D_IN), _F32),
            pltpu.VMEM((B, D_IN), _F32),
            pltpu.VMEM_SHARED((N, D_IN), _F32),
            pltpu.SemaphoreType.DMA,
        ],
    )
    def k(t_h, src_h, dst_h, z_h, out, srcbuf, dstbuf, didx, rows0, rows1,
          acc, sem):
        c = lax.axis_index("c")
        s = lax.axis_index("s")
        base = s * EPT
        pltpu.sync_copy(src_h.at[pl.ds(base, EPT)], srcbuf)
        pltpu.sync_copy(dst_h.at[pl.ds(base, EPT)], dstbuf)
        rowoff = c * N

        def addoff(kk, _):
            sl = pl.ds(kk * L, L)
            srcbuf[sl] = srcbuf[sl] + rowoff
            return ()
        lax.fori_loop(0, EPT // L, addoff, ())
        _tile_rows(s, lambda rb, nr: pltpu.sync_copy(
            z_h.at[pl.ds(rb, nr)], acc.at[pl.ds(rb, nr)]))
        plsc.subcore_barrier()

        def start_gather(i, rbuf):
            pltpu.async_copy(t_h.at[srcbuf.at[pl.ds(i * B, B)]], rbuf, sem)

        def wait_gather(rbuf):
            pltpu.make_async_copy(t_h.at[pl.ds(0, B)], rbuf, sem).wait()

        def scatter(i, rbuf):
            for j in range(B // L):
                didx[pl.ds(j * L, L)] = dstbuf[pl.ds(i * B + j * L, L)]
            pltpu.sync_copy(rbuf, acc.at[didx], add=True)

        start_gather(0, rows0)

        def pair(j, _):
            i = 2 * j
            wait_gather(rows0)
            start_gather(i + 1, rows1)
            scatter(i, rows0)
            wait_gather(rows1)
            start_gather(i + 2, rows0)
            scatter(i + 1, rows1)
            return ()
        lax.fori_loop(0, (NCH - 1) // 2, pair, ())
        wait_gather(rows0)
        scatter(NCH - 1, rows0)
        plsc.subcore_barrier()

        _tile_rows(s, lambda rb, nr: pltpu.sync_copy(
            acc.at[pl.ds(rb, nr)], out.at[pl.ds(c * N + rb, nr)]))

    return k(t2n, src, dst, zeros)


def _sc_mp1(t, src, dst, zeros):
    B = 40
    EPC = E // NC
    EPT = EPC // NS
    NCH = EPT // B
    mesh = plsc.VectorSubcoreMesh(core_axis_name="c", subcore_axis_name="s")

    @functools.partial(
        pl.kernel, mesh=mesh,
        out_type=jax.ShapeDtypeStruct((2 * N, D_IN), _F32),
        scratch_types=[
            pltpu.VMEM((EPT,), jnp.int32),
            pltpu.VMEM((B,), jnp.int32),
            pltpu.VMEM((B, D_IN), _F32),
            pltpu.VMEM((B, D_IN), _F32),
            pltpu.VMEM_SHARED((N, D_IN), _F32),
            pltpu.SemaphoreType.DMA,
        ],
    )
    def k(t_h, src_h, dst_h, z_h, out, srcbuf, didx, rows0, rows1, acc, sem):
        c = lax.axis_index("c")
        s = lax.axis_index("s")
        base = c * EPC + s * EPT
        pltpu.sync_copy(src_h.at[pl.ds(base, EPT)], srcbuf)
        _tile_rows(s, lambda rb, nr: pltpu.sync_copy(
            z_h.at[pl.ds(rb, nr)], acc.at[pl.ds(rb, nr)]))
        plsc.subcore_barrier()

        def start_gather(i, rbuf):
            pltpu.async_copy(t_h.at[srcbuf.at[pl.ds(i * B, B)]], rbuf, sem)

        def wait_gather(rbuf):
            pltpu.make_async_copy(t_h.at[pl.ds(0, B)], rbuf, sem).wait()

        def scatter(i, rbuf):
            pltpu.sync_copy(dst_h.at[pl.ds(base + i * B, B)], didx)
            pltpu.sync_copy(rbuf, acc.at[didx], add=True)

        start_gather(0, rows0)

        def pair(j, _):
            i = 2 * j
            wait_gather(rows0)
            start_gather(i + 1, rows1)
            scatter(i, rows0)
            wait_gather(rows1)
            start_gather(i + 2, rows0)
            scatter(i + 1, rows1)
            return ()
        lax.fori_loop(0, (NCH - 1) // 2, pair, ())
        wait_gather(rows0)
        scatter(NCH - 1, rows0)
        plsc.subcore_barrier()

        _tile_rows(s, lambda rb, nr: pltpu.sync_copy(
            acc.at[pl.ds(rb, nr)], out.at[pl.ds(c * N + rb, nr)]))

    return k(t, src, dst, zeros)


def _sc_edge_abs(xarr, recon, src, dst):
    B = 40
    EPC = E // NC
    EPT = EPC // NS
    NCH = EPT // B
    mesh = plsc.VectorSubcoreMesh(core_axis_name="c", subcore_axis_name="s")

    @functools.partial(
        pl.kernel, mesh=mesh,
        out_type=(jax.ShapeDtypeStruct((E, D_IN), _F32),
                  jax.ShapeDtypeStruct((E, D_IN), _F32)),
        scratch_types=[
            pltpu.VMEM((EPT,), jnp.int32),
            pltpu.VMEM((EPT,), jnp.int32),
            pltpu.VMEM((B, D_IN), _F32),
            pltpu.VMEM((B, D_IN), _F32),
            pltpu.VMEM((B, D_IN), _F32),
            pltpu.VMEM((B, D_IN), _F32),
            pltpu.VMEM((B, D_IN), _F32),
            pltpu.VMEM((B, D_IN), _F32),
            pltpu.SemaphoreType.DMA,
            pltpu.SemaphoreType.DMA,
        ],
    )
    def k(x_h, r_h, src_h, dst_h, dxi_h, dxr_h, srcbuf, dstbuf,
          ba, bb, bc, bd, bi, br, semx, semr):
        c = lax.axis_index("c")
        s = lax.axis_index("s")
        base = c * EPC + s * EPT
        pltpu.sync_copy(src_h.at[pl.ds(base, EPT)], srcbuf)
        pltpu.sync_copy(dst_h.at[pl.ds(base, EPT)], dstbuf)

        def absdiff(pa, pb, po):
            def rowbody(r, _):
                for j in range(D_IN // L):
                    sl = pl.ds(j * L, L)
                    po[r, sl] = jnp.abs(pa[r, sl] - pb[r, sl])
                return ()
            lax.fori_loop(0, B, rowbody, ())

        def body(i, _):
            off = base + i * B
            sidx = srcbuf.at[pl.ds(i * B, B)]
            didx = dstbuf.at[pl.ds(i * B, B)]
            pltpu.async_copy(x_h.at[sidx], ba, semx)
            pltpu.async_copy(x_h.at[didx], bb, semx)
            pltpu.async_copy(r_h.at[sidx], bc, semr)
            pltpu.async_copy(r_h.at[didx], bd, semr)
            pltpu.make_async_copy(x_h.at[pl.ds(0, B)], ba, semx).wait()
            pltpu.make_async_copy(x_h.at[pl.ds(0, B)], bb, semx).wait()
            absdiff(ba, bb, bi)
            pltpu.sync_copy(bi, dxi_h.at[pl.ds(off, B)])
            pltpu.make_async_copy(r_h.at[pl.ds(0, B)], bc, semr).wait()
            pltpu.make_async_copy(r_h.at[pl.ds(0, B)], bd, semr).wait()
            absdiff(bc, bd, br)
            pltpu.sync_copy(br, dxr_h.at[pl.ds(off, B)])
            return ()
        lax.fori_loop(0, NCH, body, ())

    return k(xarr, recon, src, dst)


_BN = 1000
_BE = 1000


def _rsqrt_deg(d):
    return lax.rsqrt(jnp.maximum(d, 1.0))


def _tc1_body(x_ref, w_ref, dout_ref, kt_ref, o_ref):
    cs = _rsqrt_deg(dout_ref[...])
    t = jnp.dot(x_ref[...], w_ref[...], preferred_element_type=_F32)
    t = t * (kt_ref[...] * cs)
    o_ref[0] = t[:, :D_IN]
    o_ref[1] = t[:, D_IN:]


def _tc1(x_used, w1, deg_out, kt):
    return pl.pallas_call(
        _tc1_body,
        grid=(N // _BN,),
        in_specs=[
            pl.BlockSpec((_BN, D_IN), lambda i: (i, 0)),
            pl.BlockSpec((D_IN, D_H), lambda i: (0, 0)),
            pl.BlockSpec((_BN, 1), lambda i: (i, 0)),
            pl.BlockSpec((_BN, 1), lambda i: (i, 0)),
        ],
        out_specs=pl.BlockSpec((2, _BN, D_IN), lambda i: (0, i, 0)),
        out_shape=jax.ShapeDtypeStruct((2, N, D_IN), _F32),
    )(x_used, w1, deg_out, kt)


def _tc2_body(a0_ref, a1_ref, din_ref, dout_ref, b_ref, al_ref, w_ref,
              o_ref):
    cd = _rsqrt_deg(din_ref[...])
    cs = _rsqrt_deg(dout_ref[...])
    h = jnp.concatenate([a0_ref[...], a1_ref[...]], axis=1) * cd + b_ref[...]
    a = al_ref[0, 0]
    h = jnp.where(h > 0, h, a * h)
    t = jnp.dot(h, w_ref[...], preferred_element_type=_F32) * cs
    o_ref[0] = t[:, :D_IN]
    o_ref[1] = t[:, D_IN:]


def _tc2(a2n, deg_in, deg_out, b, alpha, w):
    nb = N // _BN
    return pl.pallas_call(
        _tc2_body,
        grid=(nb,),
        in_specs=[
            pl.BlockSpec((_BN, D_IN), lambda i: (i, 0)),
            pl.BlockSpec((_BN, D_IN), lambda i: (i + N // _BN, 0)),
            pl.BlockSpec((_BN, 1), lambda i: (i, 0)),
            pl.BlockSpec((_BN, 1), lambda i: (i, 0)),
            pl.BlockSpec((1, D_H), lambda i: (0, 0)),
            pl.BlockSpec((1, 1), lambda i: (0, 0)),
            pl.BlockSpec((D_H, D_H), lambda i: (0, 0)),
        ],
        out_specs=pl.BlockSpec((2, _BN, D_IN), lambda i: (0, i, 0)),
        out_shape=jax.ShapeDtypeStruct((2, N, D_IN), _F32),
    )(a2n, a2n, deg_in, deg_out, b, alpha, w)


def _tc3_body(a0_ref, a1_ref, din_ref, dout_ref, b_ref, al_ref, we_ref,
              wd_ref, mk_ref, o_ref):
    cd = _rsqrt_deg(din_ref[...])
    cs = _rsqrt_deg(dout_ref[...])
    h = jnp.concatenate([a0_ref[...], a1_ref[...]], axis=1) * cd + b_ref[...]
    a = al_ref[0, 0]
    h = jnp.where(h > 0, h, a * h)
    rep = jnp.dot(h, we_ref[...], preferred_element_type=_F32) * mk_ref[...]
    o_ref[...] = jnp.dot(rep, wd_ref[...], preferred_element_type=_F32) * cs


def _tc3(a2n, deg_in, deg_out, b, alpha, w_e2d, w_dec, maskkeep):
    return pl.pallas_call(
        _tc3_body,
        grid=(N // _BN,),
        in_specs=[
            pl.BlockSpec((_BN, D_IN), lambda i: (i, 0)),
            pl.BlockSpec((_BN, D_IN), lambda i: (i + N // _BN, 0)),
            pl.BlockSpec((_BN, 1), lambda i: (i, 0)),
            pl.BlockSpec((_BN, 1), lambda i: (i, 0)),
            pl.BlockSpec((1, D_H), lambda i: (0, 0)),
            pl.BlockSpec((1, 1), lambda i: (0, 0)),
            pl.BlockSpec((D_H, D_H), lambda i: (0, 0)),
            pl.BlockSpec((D_H, D_IN), lambda i: (0, 0)),
            pl.BlockSpec((_BN, 1), lambda i: (i, 0)),
        ],
        out_specs=pl.BlockSpec((_BN, D_IN), lambda i: (i, 0)),
        out_shape=jax.ShapeDtypeStruct((N, D_IN), _F32),
    )(a2n, a2n, deg_in, deg_out, b, alpha, w_e2d, w_dec, maskkeep)


def _tc4_body(p0_ref, p1_ref, din_ref, b_ref, x_ref, ms_ref,
              recon_ref, acc_ref):
    i = pl.program_id(0)
    cd = _rsqrt_deg(din_ref[...])
    r = (p0_ref[...] + p1_ref[...]) * cd + b_ref[...]
    recon_ref[...] = r
    xb = x_ref[...]
    nr = jnp.sqrt(jnp.sum(r * r, axis=1, keepdims=True)) + 1e-8
    nx = jnp.sqrt(jnp.sum(xb * xb, axis=1, keepdims=True)) + 1e-8
    dots = jnp.sum(r * xb, axis=1, keepdims=True)
    term = (1.0 - dots / (nr * nx)) ** 2
    val = jnp.sum(term * ms_ref[...])
    @pl.when(i == 0)
    def _():
        acc_ref[...] = jnp.zeros((1, 1), _F32)
    acc_ref[...] = acc_ref[...] + val


def _tc4(p2n, deg_in, b_dec, x, masksel):
    return pl.pallas_call(
        _tc4_body,
        grid=(N // _BN,),
        in_specs=[
            pl.BlockSpec((_BN, D_IN), lambda i: (i, 0)),
            pl.BlockSpec((_BN, D_IN), lambda i: (i + N // _BN, 0)),
            pl.BlockSpec((_BN, 1), lambda i: (i, 0)),
            pl.BlockSpec((1, D_IN), lambda i: (0, 0)),
            pl.BlockSpec((_BN, D_IN), lambda i: (i, 0)),
            pl.BlockSpec((_BN, 1), lambda i: (i, 0)),
        ],
        out_specs=[
            pl.BlockSpec((_BN, D_IN), lambda i: (i, 0)),
            pl.BlockSpec((1, 1), lambda i: (0, 0)),
        ],
        out_shape=[
            jax.ShapeDtypeStruct((N, D_IN), _F32),
            jax.ShapeDtypeStruct((1, 1), _F32),
        ],
    )(p2n, p2n, deg_in, b_dec, x, masksel)


def _tc5_body(u_ref, v_ref, acc_ref):
    i = pl.program_id(0)
    u = u_ref[...]
    v = v_ref[...]
    mu = jnp.max(u, axis=1, keepdims=True)
    mv = jnp.max(v, axis=1, keepdims=True)
    eu = jnp.exp(u - mu)
    su = jnp.sum(eu, axis=1, keepdims=True)
    sv = jnp.sum(jnp.exp(v - mv), axis=1, keepdims=True)
    t = jnp.sum(eu * (u - v), axis=1, keepdims=True)
    row = t / su - mu - jnp.log(su) + mv + jnp.log(sv)
    val = jnp.sum(row)
    @pl.when(i == 0)
    def _():
        acc_ref[...] = jnp.zeros((1, 1), _F32)
    acc_ref[...] = acc_ref[...] + val


def _tc5(dxi, dxr):
    return pl.pallas_call(
        _tc5_body,
        grid=(E // _BE,),
        in_specs=[
            pl.BlockSpec((_BE, D_IN), lambda i: (i, 0)),
            pl.BlockSpec((_BE, D_IN), lambda i: (i, 0)),
        ],
        out_specs=pl.BlockSpec((1, 1), lambda i: (0, 0)),
        out_shape=jax.ShapeDtypeStruct((1, 1), _F32),
    )(dxi, dxr)


def kernel(x, edge_index, A, W1, b1, a1, W2, b2, a2, W_e2d, W_dec, b_dec):
    del A
    src = edge_index[0]
    dst = edge_index[1]

    mk = jax.random.key(42)
    k1, k2, k3 = jax.random.split(mk, 3)
    perm = jax.random.permutation(k1, N)
    mask_nodes = perm[:NUM_MASK]
    perm_mask = jax.random.permutation(k2, NUM_MASK)
    token_nodes = mask_nodes[perm_mask[:NUM_TOKEN]]
    noise_nodes = mask_nodes[perm_mask[-NUM_NOISE:]]
    noise_chosen = jax.random.permutation(k3, N)[:NUM_NOISE]

    ones_n = jnp.ones((N, 1), _F32)
    kt = ones_n.at[token_nodes].set(0.0)
    maskkeep = ones_n.at[mask_nodes].set(0.0)
    masksel = 1.0 - maskkeep
    x_used = x.at[noise_nodes].set(x[noise_chosen])

    zeros_rows = jnp.zeros((N, D_IN), _F32)

    deg2 = _sc_degrees(edge_index.reshape(2 * E), zeros_rows)
    deg_out = deg2[:N, :1]
    deg_in = deg2[N:, :1]

    a1_s = a1.reshape(1, 1)
    a2_s = a2.reshape(1, 1)
    b1_r = b1.reshape(1, D_H)
    b2_r = b2.reshape(1, D_H)
    bd_r = b_dec.reshape(1, D_IN)

    t2n = _tc1(x_used, W1, deg_out, kt).reshape(2 * N, D_IN)
    g2n = _sc_mp2(t2n, src, dst, zeros_rows)
    u2n = _tc2(g2n, deg_in, deg_out, b1_r, a1_s, W2).reshape(2 * N, D_IN)
    h2n = _sc_mp2(u2n, src, dst, zeros_rows)
    t3 = _tc3(h2n, deg_in, deg_out, b2_r, a2_s, W_e2d, W_dec, maskkeep)
    p2n = _sc_mp1(t3, src, dst, zeros_rows)
    recon, loss_acc = _tc4(p2n, deg_in, bd_r, x, masksel)
    loss = loss_acc[0, 0] / NUM_MASK

    dxi, dxr = _sc_edge_abs(x, recon, src, dst)
    kl_acc = _tc5(dxi, dxr)
    loss_s = kl_acc[0, 0] / E

    return (loss, loss_s, recon)


# speedup vs baseline: 5.0078x; 1.0176x over previous
import functools

import jax
import jax.numpy as jnp
from jax import lax
from jax.experimental import pallas as pl
from jax.experimental.pallas import tpu as pltpu
from jax.experimental.pallas import tpu_sc as plsc

N = 10000
E = 160000
D_IN = 128
D_H = 256
NUM_MASK = 3000
NUM_NOISE = 300
NUM_TOKEN = 2700

NC = 2
NS = 16
L = 16

_F32 = jnp.float32

_RT = 624


def _tile_rows(s, fn):
    @pl.when(s < NS - 1)
    def _():
        fn(s * _RT, _RT)
    @pl.when(s == NS - 1)
    def _():
        fn((NS - 1) * _RT, N - (NS - 1) * _RT)


def _sc_degrees(ei_flat, zeros):
    B = 80
    EPT = E // NS
    NCH = EPT // B
    mesh = plsc.VectorSubcoreMesh(core_axis_name="c", subcore_axis_name="s")

    @functools.partial(
        pl.kernel, mesh=mesh,
        out_type=jax.ShapeDtypeStruct((2 * N, D_IN), _F32),
        scratch_types=[
            pltpu.VMEM((EPT,), jnp.int32),
            pltpu.VMEM((B,), jnp.int32),
            pltpu.VMEM((B, D_IN), _F32),
            pltpu.VMEM_SHARED((N, D_IN), _F32),
        ],
    )
    def k(ei_h, z_h, out, eibuf, idxv, onesv, acc):
        c = lax.axis_index("c")
        s = lax.axis_index("s")
        pltpu.sync_copy(ei_h.at[pl.ds(c * E + s * EPT, EPT)], eibuf)
        def fill_ones(r, _):
            for j in range(D_IN // L):
                onesv[r, pl.ds(j * L, L)] = jnp.ones((L,), _F32)
            return ()
        lax.fori_loop(0, B, fill_ones, ())
        _tile_rows(s, lambda rb, nr: pltpu.sync_copy(
            z_h.at[pl.ds(rb, nr)], acc.at[pl.ds(rb, nr)]))
        plsc.subcore_barrier()

        def body(i, _):
            for j in range(B // L):
                idxv[pl.ds(j * L, L)] = eibuf[pl.ds(i * B + j * L, L)]
            pltpu.sync_copy(onesv, acc.at[idxv], add=True)
            return ()
        lax.fori_loop(0, NCH, body, ())
        plsc.subcore_barrier()

        _tile_rows(s, lambda rb, nr: pltpu.sync_copy(
            acc.at[pl.ds(rb, nr)], out.at[pl.ds(c * N + rb, nr)]))

    return k(ei_flat, zeros)


def _sc_mp2(t2n, src, dst, zeros):
    B = 80
    EPT = E // NS
    NCH = EPT // B
    mesh = plsc.VectorSubcoreMesh(core_axis_name="c", subcore_axis_name="s")

    @functools.partial(
        pl.kernel, mesh=mesh,
        out_type=jax.ShapeDtypeStruct((2 * N, D_IN), _F32),
        scratch_types=[
            pltpu.VMEM((EPT,), jnp.int32),
            pltpu.VMEM((EPT,), jnp.int32),
            pltpu.VMEM((B,), jnp.int32),
            pltpu.VMEM((B, D_IN), _F32),
            pltpu.VMEM((B, D_IN), _F32),
            pltpu.VMEM_SHARED((N, D_IN), _F32),
            pltpu.SemaphoreType.DMA,
        ],
    )
    def k(t_h, src_h, dst_h, z_h, out, srcbuf, dstbuf, didx, rows0, rows1,
          acc, sem):
        c = lax.axis_index("c")
        s = lax.axis_index("s")
        base = s * EPT
        pltpu.sync_copy(src_h.at[pl.ds(base, EPT)], srcbuf)
        pltpu.sync_copy(dst_h.at[pl.ds(base, EPT)], dstbuf)
        rowoff = c * N

        def addoff(kk, _):
            sl = pl.ds(kk * L, L)
            srcbuf[sl] = srcbuf[sl] + rowoff
            return ()
        lax.fori_loop(0, EPT // L, addoff, ())
        _tile_rows(s, lambda rb, nr: pltpu.sync_copy(
            z_h.at[pl.ds(rb, nr)], acc.at[pl.ds(rb, nr)]))
        plsc.subcore_barrier()

        def start_gather(i, rbuf):
            pltpu.async_copy(t_h.at[srcbuf.at[pl.ds(i * B, B)]], rbuf, sem)

        def wait_gather(rbuf):
            pltpu.make_async_copy(t_h.at[pl.ds(0, B)], rbuf, sem).wait()

        def scatter(i, rbuf):
            for j in range(B // L):
                didx[pl.ds(j * L, L)] = dstbuf[pl.ds(i * B + j * L, L)]
            pltpu.sync_copy(rbuf, acc.at[didx], add=True)

        start_gather(0, rows0)

        def pair(j, _):
            i = 2 * j
            wait_gather(rows0)
            start_gather(i + 1, rows1)
            scatter(i, rows0)
            wait_gather(rows1)
            start_gather(i + 2, rows0)
            scatter(i + 1, rows1)
            return ()
        lax.fori_loop(0, (NCH - 1) // 2, pair, ())
        wait_gather(rows0)
        scatter(NCH - 1, rows0)
        plsc.subcore_barrier()

        _tile_rows(s, lambda rb, nr: pltpu.sync_copy(
            acc.at[pl.ds(rb, nr)], out.at[pl.ds(c * N + rb, nr)]))

    return k(t2n, src, dst, zeros)


def _sc_mp1(t, src, dst, zeros):
    B = 40
    EPC = E // NC
    EPT = EPC // NS
    NCH = EPT // B
    mesh = plsc.VectorSubcoreMesh(core_axis_name="c", subcore_axis_name="s")

    @functools.partial(
        pl.kernel, mesh=mesh,
        out_type=jax.ShapeDtypeStruct((2 * N, D_IN), _F32),
        scratch_types=[
            pltpu.VMEM((EPT,), jnp.int32),
            pltpu.VMEM((B,), jnp.int32),
            pltpu.VMEM((B, D_IN), _F32),
            pltpu.VMEM((B, D_IN), _F32),
            pltpu.VMEM_SHARED((N, D_IN), _F32),
            pltpu.SemaphoreType.DMA,
        ],
    )
    def k(t_h, src_h, dst_h, z_h, out, srcbuf, didx, rows0, rows1, acc, sem):
        c = lax.axis_index("c")
        s = lax.axis_index("s")
        base = c * EPC + s * EPT
        pltpu.sync_copy(src_h.at[pl.ds(base, EPT)], srcbuf)
        _tile_rows(s, lambda rb, nr: pltpu.sync_copy(
            z_h.at[pl.ds(rb, nr)], acc.at[pl.ds(rb, nr)]))
        plsc.subcore_barrier()

        def start_gather(i, rbuf):
            pltpu.async_copy(t_h.at[srcbuf.at[pl.ds(i * B, B)]], rbuf, sem)

        def wait_gather(rbuf):
            pltpu.make_async_copy(t_h.at[pl.ds(0, B)], rbuf, sem).wait()

        def scatter(i, rbuf):
            pltpu.sync_copy(dst_h.at[pl.ds(base + i * B, B)], didx)
            pltpu.sync_copy(rbuf, acc.at[didx], add=True)

        start_gather(0, rows0)

        def pair(j, _):
            i = 2 * j
            wait_gather(rows0)
            start_gather(i + 1, rows1)
            scatter(i, rows0)
            wait_gather(rows1)
            start_gather(i + 2, rows0)
            scatter(i + 1, rows1)
            return ()
        lax.fori_loop(0, (NCH - 1) // 2, pair, ())
        wait_gather(rows0)
        scatter(NCH - 1, rows0)
        plsc.subcore_barrier()

        _tile_rows(s, lambda rb, nr: pltpu.sync_copy(
            acc.at[pl.ds(rb, nr)], out.at[pl.ds(c * N + rb, nr)]))

    return k(t, src, dst, zeros)


def _sc_edge_half(tab, src, dst):
    B = 40
    EPC = E // NC
    EPT = EPC // NS
    NCH = EPT // B
    mesh = plsc.VectorSubcoreMesh(core_axis_name="c", subcore_axis_name="s")

    @functools.partial(
        pl.kernel, mesh=mesh,
        out_type=jax.ShapeDtypeStruct((E, D_IN), _F32),
        scratch_types=[
            pltpu.VMEM((EPT,), jnp.int32),
            pltpu.VMEM((EPT,), jnp.int32),
            pltpu.VMEM((B, D_IN), _F32),
            pltpu.VMEM((B, D_IN), _F32),
            pltpu.VMEM((B, D_IN), _F32),
            pltpu.VMEM((B, D_IN), _F32),
            pltpu.SemaphoreType.DMA,
            pltpu.SemaphoreType.DMA,
        ],
    )
    def k(t_h, src_h, dst_h, out, srcbuf, dstbuf, a0, b0, a1, b1,
          sem0, sem1):
        c = lax.axis_index("c")
        s = lax.axis_index("s")
        base = c * EPC + s * EPT
        pltpu.sync_copy(src_h.at[pl.ds(base, EPT)], srcbuf)
        pltpu.sync_copy(dst_h.at[pl.ds(base, EPT)], dstbuf)

        bufs = ((a0, b0, sem0), (a1, b1, sem1))

        def start(i, ph):
            pa, pb, sem = bufs[ph]
            pltpu.async_copy(t_h.at[srcbuf.at[pl.ds(i * B, B)]], pa, sem)
            pltpu.async_copy(t_h.at[dstbuf.at[pl.ds(i * B, B)]], pb, sem)

        def wait(ph):
            pa, pb, sem = bufs[ph]
            pltpu.make_async_copy(t_h.at[pl.ds(0, B)], pa, sem).wait()
            pltpu.make_async_copy(t_h.at[pl.ds(0, B)], pb, sem).wait()

        def compute(i, ph):
            pa, pb, _ = bufs[ph]

            def rowbody(r, _):
                for j in range(D_IN // L):
                    sl = pl.ds(j * L, L)
                    pa[r, sl] = jnp.abs(pa[r, sl] - pb[r, sl])
                return ()
            lax.fori_loop(0, B, rowbody, ())
            pltpu.sync_copy(pa, out.at[pl.ds(base + i * B, B)])

        start(0, 0)

        def pair(jj, _):
            i = 2 * jj
            wait(0)
            start(i + 1, 1)
            compute(i, 0)
            wait(1)
            start(i + 2, 0)
            compute(i + 1, 1)
            return ()
        lax.fori_loop(0, (NCH - 1) // 2, pair, ())
        wait(0)
        compute(NCH - 1, 0)

    return k(tab, src, dst)


_BN = 1000
_BE = 1000


def _rsqrt_deg(d):
    return lax.rsqrt(jnp.maximum(d, 1.0))


def _tc1_body(x_ref, w_ref, dout_ref, kt_ref, o_ref):
    cs = _rsqrt_deg(dout_ref[...])
    t = jnp.dot(x_ref[...], w_ref[...], preferred_element_type=_F32)
    t = t * (kt_ref[...] * cs)
    o_ref[0] = t[:, :D_IN]
    o_ref[1] = t[:, D_IN:]


def _tc1(x_used, w1, deg_out, kt):
    return pl.pallas_call(
        _tc1_body,
        grid=(N // _BN,),
        in_specs=[
            pl.BlockSpec((_BN, D_IN), lambda i: (i, 0)),
            pl.BlockSpec((D_IN, D_H), lambda i: (0, 0)),
            pl.BlockSpec((_BN, 1), lambda i: (i, 0)),
            pl.BlockSpec((_BN, 1), lambda i: (i, 0)),
        ],
        out_specs=pl.BlockSpec((2, _BN, D_IN), lambda i: (0, i, 0)),
        out_shape=jax.ShapeDtypeStruct((2, N, D_IN), _F32),
    )(x_used, w1, deg_out, kt)


def _tc2_body(a0_ref, a1_ref, din_ref, dout_ref, b_ref, al_ref, w_ref,
              o_ref):
    cd = _rsqrt_deg(din_ref[...])
    cs = _rsqrt_deg(dout_ref[...])
    h = jnp.concatenate([a0_ref[...], a1_ref[...]], axis=1) * cd + b_ref[...]
    a = al_ref[0, 0]
    h = jnp.where(h > 0, h, a * h)
    t = jnp.dot(h, w_ref[...], preferred_element_type=_F32) * cs
    o_ref[0] = t[:, :D_IN]
    o_ref[1] = t[:, D_IN:]


def _tc2(a2n, deg_in, deg_out, b, alpha, w):
    nb = N // _BN
    return pl.pallas_call(
        _tc2_body,
        grid=(nb,),
        in_specs=[
            pl.BlockSpec((_BN, D_IN), lambda i: (i, 0)),
            pl.BlockSpec((_BN, D_IN), lambda i: (i + N // _BN, 0)),
            pl.BlockSpec((_BN, 1), lambda i: (i, 0)),
            pl.BlockSpec((_BN, 1), lambda i: (i, 0)),
            pl.BlockSpec((1, D_H), lambda i: (0, 0)),
            pl.BlockSpec((1, 1), lambda i: (0, 0)),
            pl.BlockSpec((D_H, D_H), lambda i: (0, 0)),
        ],
        out_specs=pl.BlockSpec((2, _BN, D_IN), lambda i: (0, i, 0)),
        out_shape=jax.ShapeDtypeStruct((2, N, D_IN), _F32),
    )(a2n, a2n, deg_in, deg_out, b, alpha, w)


def _tc3_body(a0_ref, a1_ref, din_ref, dout_ref, b_ref, al_ref, we_ref,
              wd_ref, mk_ref, o_ref):
    cd = _rsqrt_deg(din_ref[...])
    cs = _rsqrt_deg(dout_ref[...])
    h = jnp.concatenate([a0_ref[...], a1_ref[...]], axis=1) * cd + b_ref[...]
    a = al_ref[0, 0]
    h = jnp.where(h > 0, h, a * h)
    rep = jnp.dot(h, we_ref[...], preferred_element_type=_F32) * mk_ref[...]
    o_ref[...] = jnp.dot(rep, wd_ref[...], preferred_element_type=_F32) * cs


def _tc3(a2n, deg_in, deg_out, b, alpha, w_e2d, w_dec, maskkeep):
    return pl.pallas_call(
        _tc3_body,
        grid=(N // _BN,),
        in_specs=[
            pl.BlockSpec((_BN, D_IN), lambda i: (i, 0)),
            pl.BlockSpec((_BN, D_IN), lambda i: (i + N // _BN, 0)),
            pl.BlockSpec((_BN, 1), lambda i: (i, 0)),
            pl.BlockSpec((_BN, 1), lambda i: (i, 0)),
            pl.BlockSpec((1, D_H), lambda i: (0, 0)),
            pl.BlockSpec((1, 1), lambda i: (0, 0)),
            pl.BlockSpec((D_H, D_H), lambda i: (0, 0)),
            pl.BlockSpec((D_H, D_IN), lambda i: (0, 0)),
            pl.BlockSpec((_BN, 1), lambda i: (i, 0)),
        ],
        out_specs=pl.BlockSpec((_BN, D_IN), lambda i: (i, 0)),
        out_shape=jax.ShapeDtypeStruct((N, D_IN), _F32),
    )(a2n, a2n, deg_in, deg_out, b, alpha, w_e2d, w_dec, maskkeep)


def _tc4_body(p0_ref, p1_ref, din_ref, b_ref, x_ref, ms_ref,
              recon_ref, acc_ref):
    i = pl.program_id(0)
    cd = _rsqrt_deg(din_ref[...])
    r = (p0_ref[...] + p1_ref[...]) * cd + b_ref[...]
    recon_ref[...] = r
    xb = x_ref[...]
    nr = jnp.sqrt(jnp.sum(r * r, axis=1, keepdims=True)) + 1e-8
    nx = jnp.sqrt(jnp.sum(xb * xb, axis=1, keepdims=True)) + 1e-8
    dots = jnp.sum(r * xb, axis=1, keepdims=True)
    term = (1.0 - dots / (nr * nx)) ** 2
    val = jnp.sum(term * ms_ref[...])
    @pl.when(i == 0)
    def _():
        acc_ref[...] = jnp.zeros((1, 1), _F32)
    acc_ref[...] = acc_ref[...] + val


def _tc4(p2n, deg_in, b_dec, x, masksel):
    return pl.pallas_call(
        _tc4_body,
        grid=(N // _BN,),
        in_specs=[
            pl.BlockSpec((_BN, D_IN), lambda i: (i, 0)),
            pl.BlockSpec((_BN, D_IN), lambda i: (i + N // _BN, 0)),
            pl.BlockSpec((_BN, 1), lambda i: (i, 0)),
            pl.BlockSpec((1, D_IN), lambda i: (0, 0)),
            pl.BlockSpec((_BN, D_IN), lambda i: (i, 0)),
            pl.BlockSpec((_BN, 1), lambda i: (i, 0)),
        ],
        out_specs=[
            pl.BlockSpec((_BN, D_IN), lambda i: (i, 0)),
            pl.BlockSpec((1, 1), lambda i: (0, 0)),
        ],
        out_shape=[
            jax.ShapeDtypeStruct((N, D_IN), _F32),
            jax.ShapeDtypeStruct((1, 1), _F32),
        ],
    )(p2n, p2n, deg_in, b_dec, x, masksel)


def _tc5_body(u_ref, v_ref, acc_ref):
    i = pl.program_id(0)
    u = u_ref[...]
    v = v_ref[...]
    mu = jnp.max(u, axis=1, keepdims=True)
    mv = jnp.max(v, axis=1, keepdims=True)
    eu = jnp.exp(u - mu)
    su = jnp.sum(eu, axis=1, keepdims=True)
    sv = jnp.sum(jnp.exp(v - mv), axis=1, keepdims=True)
    t = jnp.sum(eu * (u - v), axis=1, keepdims=True)
    row = t / su - mu - jnp.log(su) + mv + jnp.log(sv)
    val = jnp.sum(row)
    @pl.when(i == 0)
    def _():
        acc_ref[...] = jnp.zeros((1, 1), _F32)
    acc_ref[...] = acc_ref[...] + val


def _tc5(dxi, dxr):
    return pl.pallas_call(
        _tc5_body,
        grid=(E // _BE,),
        in_specs=[
            pl.BlockSpec((_BE, D_IN), lambda i: (i, 0)),
            pl.BlockSpec((_BE, D_IN), lambda i: (i, 0)),
        ],
        out_specs=pl.BlockSpec((1, 1), lambda i: (0, 0)),
        out_shape=jax.ShapeDtypeStruct((1, 1), _F32),
    )(dxi, dxr)


def kernel(x, edge_index, A, W1, b1, a1, W2, b2, a2, W_e2d, W_dec, b_dec):
    del A
    src = edge_index[0]
    dst = edge_index[1]

    mk = jax.random.key(42)
    k1, k2, k3 = jax.random.split(mk, 3)
    perm = jax.random.permutation(k1, N)
    mask_nodes = perm[:NUM_MASK]
    perm_mask = jax.random.permutation(k2, NUM_MASK)
    token_nodes = mask_nodes[perm_mask[:NUM_TOKEN]]
    noise_nodes = mask_nodes[perm_mask[-NUM_NOISE:]]
    noise_chosen = jax.random.permutation(k3, N)[:NUM_NOISE]

    ones_n = jnp.ones((N, 1), _F32)
    kt = ones_n.at[token_nodes].set(0.0)
    maskkeep = ones_n.at[mask_nodes].set(0.0)
    masksel = 1.0 - maskkeep
    x_used = x.at[noise_nodes].set(x[noise_chosen])

    zeros_rows = jnp.zeros((N, D_IN), _F32)

    deg2 = _sc_degrees(edge_index.reshape(2 * E), zeros_rows)
    deg_out = deg2[:N, :1]
    deg_in = deg2[N:, :1]

    dxi = _sc_edge_half(x, src, dst)

    a1_s = a1.reshape(1, 1)
    a2_s = a2.reshape(1, 1)
    b1_r = b1.reshape(1, D_H)
    b2_r = b2.reshape(1, D_H)
    bd_r = b_dec.reshape(1, D_IN)

    t2n = _tc1(x_used, W1, deg_out, kt).reshape(2 * N, D_IN)
    g2n = _sc_mp2(t2n, src, dst, zeros_rows)
    u2n = _tc2(g2n, deg_in, deg_out, b1_r, a1_s, W2).reshape(2 * N, D_IN)
    h2n = _sc_mp2(u2n, src, dst, zeros_rows)
    t3 = _tc3(h2n, deg_in, deg_out, b2_r, a2_s, W_e2d, W_dec, maskkeep)
    p2n = _sc_mp1(t3, src, dst, zeros_rows)
    recon, loss_acc = _tc4(p2n, deg_in, bd_r, x, masksel)
    loss = loss_acc[0, 0] / NUM_MASK

    dxr = _sc_edge_half(recon, src, dst)
    kl_acc = _tc5(dxi, dxr)
    loss_s = kl_acc[0, 0] / E

    return (loss, loss_s, recon)
